# baseline (speedup 1.0000x reference)
import functools

import jax
import jax.numpy as jnp
from jax import lax
from jax.experimental import pallas as pl
from jax.experimental.pallas import tpu as pltpu
from jax.experimental.pallas import tpu_sc as plsc

N_ATOMS = 10000
N_EDGE = 320000
N_UND = 160000
ATOM_DIM = 128
BOND_DIM = 16

NC = 2
NS = 16
NW = NC * NS
CHUNK = 40
NCHUNK = N_EDGE // (NW * CHUNK)
NSEC = 5
SCHUNK = NCHUNK // NSEC
GCH = 80
GNCHUNK = N_EDGE // (NW * GCH)
NSLICE = 5
GSCH = GNCHUNK // NSLICE
ACC_ROWS = 10240
ROWS_PER_TILE = ACC_ROWS // NS


def _gather_body(nsec, acat, ncat, bcat, src_r, dst_r, u_r, h_out,
                 sidx, didx, uidx, a0, n0, b0, a1, n1, b1,
                 g0, g1, s0, s1):
    wid = lax.axis_index("s") * NC + lax.axis_index("c")
    row0 = wid * (nsec * GSCH)
    bufs = ((a0, n0, b0, g0, s0), (a1, n1, b1, g1, s1))

    def start_gathers(kk, p):
        a, n, b, g, _ = bufs[p]
        pltpu.async_copy(acat.at[sidx.at[kk]], a, g)
        pltpu.async_copy(ncat.at[didx.at[kk]], n, g)
        pltpu.async_copy(bcat.at[uidx.at[kk]], b, g)

    def wait_gathers(p):
        a, n, b, g, _ = bufs[p]
        pltpu.make_async_copy(acat.at[sidx.at[0]], a, g).wait()
        pltpu.make_async_copy(ncat.at[didx.at[0]], n, g).wait()
        pltpu.make_async_copy(bcat.at[uidx.at[0]], b, g).wait()

    def compute_store(kk, p, sec):
        a, n, b, _, s = bufs[p]

        @plsc.parallel_loop(0, GCH, step=1, unroll=4)
        def _row(r):
            for j in range(ATOM_DIM // 16):
                sl = pl.ds(j * 16, 16)
                plsc.addupdate(a.at[r, sl], n[r, sl])
                plsc.addupdate(a.at[r, sl], b[r, sl])
        e0 = (row0 + sec * GSCH + kk) * GCH
        pltpu.async_copy(a, h_out.at[pl.ds(e0, GCH)], s)

    def wait_store(p):
        a, _, _, _, s = bufs[p]
        pltpu.make_async_copy(a, h_out.at[pl.ds(0, GCH)], s).wait()

    def section(sec, carry):
        pltpu.sync_copy(src_r.at[wid, sec], sidx)
        pltpu.sync_copy(dst_r.at[wid, sec], didx)
        pltpu.sync_copy(u_r.at[wid, sec], uidx)
        start_gathers(0, 0)

        def pair(kp, c1):
            kk = kp * 2
            wait_gathers(0)
            pl.when(kp > 0)(lambda: wait_store(1))
            start_gathers(kk + 1, 1)
            compute_store(kk, 0, sec)
            wait_gathers(1)
            wait_store(0)
            start_gathers(kk + 2, 0)
            compute_store(kk + 1, 1, sec)
            return c1

        lax.fori_loop(0, GSCH // 2, pair, 0)
        wait_gathers(0)
        wait_store(1)
        compute_store(GSCH - 1, 0, sec)
        wait_store(0)
        return carry

    lax.fori_loop(0, nsec, section, 0)


def _sc_gather(acat, ncat, bcat, src_r, dst_r, u_r):
    nsec = src_r.shape[1]
    n_edges = NW * nsec * GSCH * GCH
    mesh = plsc.VectorSubcoreMesh(core_axis_name="c", subcore_axis_name="s")
    f = pl.kernel(
        functools.partial(_gather_body, nsec),
        out_type=jax.ShapeDtypeStruct((n_edges, ATOM_DIM), jnp.float32),
        mesh=mesh,
        scratch_types=[
            pltpu.VMEM((GSCH, GCH), jnp.int32),
            pltpu.VMEM((GSCH, GCH), jnp.int32),
            pltpu.VMEM((GSCH, GCH), jnp.int32),
            pltpu.VMEM((GCH, ATOM_DIM), jnp.float32),
            pltpu.VMEM((GCH, ATOM_DIM), jnp.float32),
            pltpu.VMEM((GCH, ATOM_DIM), jnp.float32),
            pltpu.VMEM((GCH, ATOM_DIM), jnp.float32),
            pltpu.VMEM((GCH, ATOM_DIM), jnp.float32),
            pltpu.VMEM((GCH, ATOM_DIM), jnp.float32),
            pltpu.SemaphoreType.DMA,
            pltpu.SemaphoreType.DMA,
            pltpu.SemaphoreType.DMA,
            pltpu.SemaphoreType.DMA,
        ],
    )
    return f(acat, ncat, bcat, src_r, dst_r, u_r)


def _scatter_body(mr0, mr1, mr2, mr3, mr4, bw, src_r, u_r, out2,
                  sidx, uidx, m0, w0, m1, w1, acc,
                  g0, g1, t0, t1):
    c = lax.axis_index("c")
    s = lax.axis_index("s")
    wid = s * NC + c
    row0 = wid * SCHUNK
    mraws = (mr0, mr1, mr2, mr3, mr4)
    bufs = ((m0, w0, g0, t0), (m1, w1, g1, t1))

    def zrow(r, carry):
        for j in range(ATOM_DIM // 16):
            m0[r, pl.ds(j * 16, 16)] = jnp.zeros((16,), jnp.float32)
        return carry

    lax.fori_loop(0, CHUNK, zrow, 0)
    for j in range(ROWS_PER_TILE // CHUNK):
        pltpu.sync_copy(m0, acc.at[pl.ds(s * ROWS_PER_TILE + j * CHUNK, CHUNK)])
    plsc.subcore_barrier()

    def start_loads(kk, p, mr):
        m, w, g, _ = bufs[p]
        e0 = (row0 + kk) * CHUNK
        pltpu.async_copy(mr.at[pl.ds(e0, CHUNK)], m, g)
        pltpu.async_copy(bw.at[uidx.at[kk]], w, g)

    def wait_loads(p):
        m, w, g, _ = bufs[p]
        pltpu.make_async_copy(mr0.at[pl.ds(0, CHUNK)], m, g).wait()
        pltpu.make_async_copy(bw.at[uidx.at[0]], w, g).wait()

    def compute_scatter(kk, p):
        m, w, _, t = bufs[p]

        @plsc.parallel_loop(0, CHUNK, step=1, unroll=4)
        def _row(r):
            for j in range(ATOM_DIM // 16):
                sl = pl.ds(j * 16, 16)
                m[r, sl] = m[r, sl] * w[r, sl]
        pltpu.async_copy(m, acc.at[sidx.at[kk]], t, add=True)

    def wait_scatter(p):
        m, _, _, t = bufs[p]
        pltpu.make_async_copy(m, acc.at[sidx.at[0]], t).wait()

    for sec in range(NSEC):
        mr = mraws[sec]
        pltpu.sync_copy(src_r.at[wid, sec], sidx)
        pltpu.sync_copy(u_r.at[wid, sec], uidx)
        start_loads(0, 0, mr)

        def pair(kp, c1, mr=mr):
            kk = kp * 2
            wait_loads(0)
            pl.when(kp > 0)(lambda: wait_scatter(1))
            start_loads(kk + 1, 1, mr)
            compute_scatter(kk, 0)
            wait_loads(1)
            wait_scatter(0)
            pl.when(kp < SCHUNK // 2 - 1)(lambda: start_loads(kk + 2, 0, mr))
            compute_scatter(kk + 1, 1)
            return c1

        lax.fori_loop(0, SCHUNK // 2, pair, 0)
        wait_scatter(1)

    plsc.subcore_barrier()
    for j in range(ROWS_PER_TILE // CHUNK):
        r0 = s * ROWS_PER_TILE + j * CHUNK
        pltpu.sync_copy(acc.at[pl.ds(r0, CHUNK)], m0)
        pltpu.sync_copy(m0, out2.at[c, pl.ds(r0, CHUNK)])


def _sc_scatter(mraws, bond_weights, src_r, u_r):
    mesh = plsc.VectorSubcoreMesh(core_axis_name="c", subcore_axis_name="s")
    f = pl.kernel(
        _scatter_body,
        out_type=jax.ShapeDtypeStruct((NC, ACC_ROWS, ATOM_DIM), jnp.float32),
        mesh=mesh,
        scratch_types=[
            pltpu.VMEM((SCHUNK, CHUNK), jnp.int32),
            pltpu.VMEM((SCHUNK, CHUNK), jnp.int32),
            pltpu.VMEM((CHUNK, ATOM_DIM), jnp.float32),
            pltpu.VMEM((CHUNK, ATOM_DIM), jnp.float32),
            pltpu.VMEM((CHUNK, ATOM_DIM), jnp.float32),
            pltpu.VMEM((CHUNK, ATOM_DIM), jnp.float32),
            pltpu.VMEM_SHARED((ACC_ROWS, ATOM_DIM), jnp.float32),
            pltpu.SemaphoreType.DMA,
            pltpu.SemaphoreType.DMA,
            pltpu.SemaphoreType.DMA,
            pltpu.SemaphoreType.DMA,
        ],
    )
    return f(*mraws, bond_weights, src_r, u_r)


def _prep_body(a_ref, wa_ref, wn_ref, oa_ref, on_ref):
    a = a_ref[...]
    oa_ref[...] = jnp.dot(a, wa_ref[...], preferred_element_type=jnp.float32)
    on_ref[...] = jnp.dot(a, wn_ref[...], preferred_element_type=jnp.float32)


def _tc_prep(atom_feas, wa, wn):
    blk = 2000
    grid = N_ATOMS // blk
    return pl.pallas_call(
        _prep_body,
        grid=(grid,),
        in_specs=[
            pl.BlockSpec((blk, ATOM_DIM), lambda i: (i, 0)),
            pl.BlockSpec((ATOM_DIM, ATOM_DIM), lambda i: (0, 0)),
            pl.BlockSpec((ATOM_DIM, ATOM_DIM), lambda i: (0, 0)),
        ],
        out_specs=[
            pl.BlockSpec((blk, ATOM_DIM), lambda i: (i, 0)),
            pl.BlockSpec((blk, ATOM_DIM), lambda i: (i, 0)),
        ],
        out_shape=[
            jax.ShapeDtypeStruct((N_ATOMS, ATOM_DIM), jnp.float32),
            jax.ShapeDtypeStruct((N_ATOMS, ATOM_DIM), jnp.float32),
        ],
    )(atom_feas, wa, wn)


def _bond_body(b_ref, wb_ref, b1_ref, o_ref):
    o_ref[...] = (jnp.dot(b_ref[...], wb_ref[...],
                          preferred_element_type=jnp.float32) + b1_ref[...])


def _tc_prep_bond(bond_feas, wb, b1):
    blk = 4000
    grid = N_UND // blk
    return pl.pallas_call(
        _bond_body,
        grid=(grid,),
        in_specs=[
            pl.BlockSpec((blk, BOND_DIM), lambda i: (i, 0)),
            pl.BlockSpec((BOND_DIM, ATOM_DIM), lambda i: (0, 0)),
            pl.BlockSpec((1, ATOM_DIM), lambda i: (0, 0)),
        ],
        out_specs=pl.BlockSpec((blk, ATOM_DIM), lambda i: (i, 0)),
        out_shape=jax.ShapeDtypeStruct((N_UND, ATOM_DIM), jnp.float32),
    )(bond_feas, wb, b1)


def _mlp_body(h_ref, w2c_ref, b2c_ref, w2g_ref, b2g_ref, o_ref):
    h = h_ref[...]
    s = (h * jax.nn.sigmoid(h)).astype(jnp.bfloat16)
    core = jnp.dot(s[:, :64], w2c_ref[...],
                   preferred_element_type=jnp.float32) + b2c_ref[...]
    gate = jnp.dot(s[:, 64:], w2g_ref[...],
                   preferred_element_type=jnp.float32) + b2g_ref[...]
    o_ref[...] = core * jax.nn.sigmoid(core) * jax.nn.sigmoid(gate)


def _tc_mlp(h_sum, w2c, b2c, w2g, b2g):
    n = h_sum.shape[0]
    blk = 3200
    grid = n // blk
    return pl.pallas_call(
        _mlp_body,
        grid=(grid,),
        in_specs=[
            pl.BlockSpec((blk, ATOM_DIM), lambda i: (i, 0)),
            pl.BlockSpec((64, ATOM_DIM), lambda i: (0, 0)),
            pl.BlockSpec((1, ATOM_DIM), lambda i: (0, 0)),
            pl.BlockSpec((64, ATOM_DIM), lambda i: (0, 0)),
            pl.BlockSpec((1, ATOM_DIM), lambda i: (0, 0)),
        ],
        out_specs=pl.BlockSpec((blk, ATOM_DIM), lambda i: (i, 0)),
        out_shape=jax.ShapeDtypeStruct((n, ATOM_DIM), jnp.float32),
    )(h_sum, w2c, b2c, w2g, b2g)


def _out_body(p_ref, w_ref, b_ref, a_ref, o_ref):
    p = p_ref[0] + p_ref[1]
    o_ref[...] = (jnp.dot(p, w_ref[...], preferred_element_type=jnp.float32)
                  + b_ref[...] + a_ref[...])


def _tc_out(partials, wout, bout, atom_feas):
    blk = 2000
    grid = N_ATOMS // blk
    return pl.pallas_call(
        _out_body,
        grid=(grid,),
        in_specs=[
            pl.BlockSpec((NC, blk, ATOM_DIM), lambda i: (0, i, 0)),
            pl.BlockSpec((ATOM_DIM, ATOM_DIM), lambda i: (0, 0)),
            pl.BlockSpec((1, ATOM_DIM), lambda i: (0, 0)),
            pl.BlockSpec((blk, ATOM_DIM), lambda i: (i, 0)),
        ],
        out_specs=pl.BlockSpec((blk, ATOM_DIM), lambda i: (i, 0)),
        out_shape=jax.ShapeDtypeStruct((N_ATOMS, ATOM_DIM), jnp.float32),
    )(partials, wout, bout, atom_feas)


def kernel(atom_feas, bond_feas, bond_weights, atom_graph, directed2undirected,
           W1c, b1c, W2c, b2c, W1g, b1g, W2g, b2g, Wout, bout):
    wa = jnp.concatenate([W1c[:ATOM_DIM], W1g[:ATOM_DIM]], axis=1)
    wn = jnp.concatenate([W1c[ATOM_DIM + BOND_DIM:], W1g[ATOM_DIM + BOND_DIM:]], axis=1)
    wb = jnp.concatenate([W1c[ATOM_DIM:ATOM_DIM + BOND_DIM],
                          W1g[ATOM_DIM:ATOM_DIM + BOND_DIM]], axis=1)
    b1 = jnp.concatenate([b1c, b1g]).reshape(1, ATOM_DIM)

    src = atom_graph[:, 0]
    dst = atom_graph[:, 1]
    src_r = src.reshape(NW, NSEC, SCHUNK, CHUNK)
    u_r = directed2undirected.reshape(NW, NSEC, SCHUNK, CHUNK)
    src_g = src.reshape(NW, NSLICE, GSCH, GCH)
    dst_g = dst.reshape(NW, NSLICE, GSCH, GCH)
    u_g = directed2undirected.reshape(NW, NSLICE, GSCH, GCH)

    acat, ncat = _tc_prep(atom_feas, wa, wn)
    bcat = _tc_prep_bond(bond_feas, wb, b1)
    w2cb = W2c.astype(jnp.bfloat16)
    w2gb = W2g.astype(jnp.bfloat16)
    b2cr = b2c.reshape(1, -1)
    b2gr = b2g.reshape(1, -1)
    mraws = []
    for si in range(NSLICE):
        h_si = _sc_gather(acat, ncat, bcat,
                          src_g[:, si:si + 1], dst_g[:, si:si + 1],
                          u_g[:, si:si + 1])
        mraws.append(_tc_mlp(h_si, w2cb, b2cr, w2gb, b2gr))
    partials = _sc_scatter(mraws, bond_weights, src_r, u_r)
    return _tc_out(partials, Wout, bout.reshape(1, -1), atom_feas)


# speedup vs baseline: 1.0053x; 1.0053x over previous
import functools

import jax
import jax.numpy as jnp
from jax import lax
from jax.experimental import pallas as pl
from jax.experimental.pallas import tpu as pltpu
from jax.experimental.pallas import tpu_sc as plsc

N_ATOMS = 10000
N_EDGE = 320000
N_UND = 160000
ATOM_DIM = 128
BOND_DIM = 16

NC = 2
NS = 16
NW = NC * NS
CHUNK = 40
NCHUNK = N_EDGE // (NW * CHUNK)
NSEC = 5
SCHUNK = NCHUNK // NSEC
GCH = 80
GNCHUNK = N_EDGE // (NW * GCH)
NSLICE = 5
GSCH = GNCHUNK // NSLICE
ACC_ROWS = 10240
ROWS_PER_TILE = ACC_ROWS // NS


def _gather_body(nsec, acat, ncat, bcat, src_r, dst_r, u_r, h_out,
                 sidx, didx, uidx, a0, n0, b0, a1, n1, b1,
                 g0, g1, s0, s1):
    wid = lax.axis_index("s") * NC + lax.axis_index("c")
    row0 = wid * (nsec * GSCH)
    bufs = ((a0, n0, b0, g0, s0), (a1, n1, b1, g1, s1))

    def start_gathers(kk, p):
        a, n, b, g, _ = bufs[p]
        pltpu.async_copy(acat.at[sidx.at[kk]], a, g)
        pltpu.async_copy(ncat.at[didx.at[kk]], n, g)
        pltpu.async_copy(bcat.at[uidx.at[kk]], b, g)

    def wait_gathers(p):
        a, n, b, g, _ = bufs[p]
        pltpu.make_async_copy(acat.at[sidx.at[0]], a, g).wait()
        pltpu.make_async_copy(ncat.at[didx.at[0]], n, g).wait()
        pltpu.make_async_copy(bcat.at[uidx.at[0]], b, g).wait()

    def compute_store(kk, p, sec):
        a, n, b, _, s = bufs[p]

        def row(r, c2):
            for j in range(ATOM_DIM // 16):
                sl = pl.ds(j * 16, 16)
                plsc.addupdate(a.at[r, sl], n[r, sl])
                plsc.addupdate(a.at[r, sl], b[r, sl])
            return c2

        lax.fori_loop(0, GCH, row, 0)
        e0 = (row0 + sec * GSCH + kk) * GCH
        pltpu.async_copy(a, h_out.at[pl.ds(e0, GCH)], s)

    def wait_store(p):
        a, _, _, _, s = bufs[p]
        pltpu.make_async_copy(a, h_out.at[pl.ds(0, GCH)], s).wait()

    def section(sec, carry):
        pltpu.sync_copy(src_r.at[wid, sec], sidx)
        pltpu.sync_copy(dst_r.at[wid, sec], didx)
        pltpu.sync_copy(u_r.at[wid, sec], uidx)
        start_gathers(0, 0)

        def pair(kp, c1):
            kk = kp * 2
            wait_gathers(0)
            pl.when(kp > 0)(lambda: wait_store(1))
            start_gathers(kk + 1, 1)
            compute_store(kk, 0, sec)
            wait_gathers(1)
            wait_store(0)
            start_gathers(kk + 2, 0)
            compute_store(kk + 1, 1, sec)
            return c1

        lax.fori_loop(0, GSCH // 2, pair, 0)
        wait_gathers(0)
        wait_store(1)
        compute_store(GSCH - 1, 0, sec)
        wait_store(0)
        return carry

    lax.fori_loop(0, nsec, section, 0)


def _sc_gather(acat, ncat, bcat, src_r, dst_r, u_r):
    nsec = src_r.shape[1]
    n_edges = NW * nsec * GSCH * GCH
    mesh = plsc.VectorSubcoreMesh(core_axis_name="c", subcore_axis_name="s")
    f = pl.kernel(
        functools.partial(_gather_body, nsec),
        out_type=jax.ShapeDtypeStruct((n_edges, ATOM_DIM), jnp.float32),
        mesh=mesh,
        scratch_types=[
            pltpu.VMEM((GSCH, GCH), jnp.int32),
            pltpu.VMEM((GSCH, GCH), jnp.int32),
            pltpu.VMEM((GSCH, GCH), jnp.int32),
            pltpu.VMEM((GCH, ATOM_DIM), jnp.float32),
            pltpu.VMEM((GCH, ATOM_DIM), jnp.float32),
            pltpu.VMEM((GCH, ATOM_DIM), jnp.float32),
            pltpu.VMEM((GCH, ATOM_DIM), jnp.float32),
            pltpu.VMEM((GCH, ATOM_DIM), jnp.float32),
            pltpu.VMEM((GCH, ATOM_DIM), jnp.float32),
            pltpu.SemaphoreType.DMA,
            pltpu.SemaphoreType.DMA,
            pltpu.SemaphoreType.DMA,
            pltpu.SemaphoreType.DMA,
        ],
    )
    return f(acat, ncat, bcat, src_r, dst_r, u_r)


def _scatter_body(mr0, mr1, mr2, mr3, mr4, bw, src_r, u_r, out2,
                  sidx, uidx, m0, w0, m1, w1, acc,
                  g0, g1, t0, t1):
    c = lax.axis_index("c")
    s = lax.axis_index("s")
    wid = s * NC + c
    row0 = wid * SCHUNK
    mraws = (mr0, mr1, mr2, mr3, mr4)
    bufs = ((m0, w0, g0, t0), (m1, w1, g1, t1))

    def zrow(r, carry):
        for j in range(ATOM_DIM // 16):
            m0[r, pl.ds(j * 16, 16)] = jnp.zeros((16,), jnp.float32)
        return carry

    lax.fori_loop(0, CHUNK, zrow, 0)
    for j in range(ROWS_PER_TILE // CHUNK):
        pltpu.sync_copy(m0, acc.at[pl.ds(s * ROWS_PER_TILE + j * CHUNK, CHUNK)])
    plsc.subcore_barrier()

    def start_loads(kk, p, mr):
        m, w, g, _ = bufs[p]
        e0 = (row0 + kk) * CHUNK
        pltpu.async_copy(mr.at[pl.ds(e0, CHUNK)], m, g)
        pltpu.async_copy(bw.at[uidx.at[kk]], w, g)

    def wait_loads(p):
        m, w, g, _ = bufs[p]
        pltpu.make_async_copy(mr0.at[pl.ds(0, CHUNK)], m, g).wait()
        pltpu.make_async_copy(bw.at[uidx.at[0]], w, g).wait()

    def compute_scatter(kk, p):
        m, w, _, t = bufs[p]

        def row(r, c2):
            for j in range(ATOM_DIM // 16):
                sl = pl.ds(j * 16, 16)
                m[r, sl] = m[r, sl] * w[r, sl]
            return c2

        lax.fori_loop(0, CHUNK, row, 0)
        pltpu.async_copy(m, acc.at[sidx.at[kk]], t, add=True)

    def wait_scatter(p):
        m, _, _, t = bufs[p]
        pltpu.make_async_copy(m, acc.at[sidx.at[0]], t).wait()

    for sec in range(NSEC):
        mr = mraws[sec]
        pltpu.sync_copy(src_r.at[wid, sec], sidx)
        pltpu.sync_copy(u_r.at[wid, sec], uidx)
        start_loads(0, 0, mr)

        def pair(kp, c1, mr=mr):
            kk = kp * 2
            wait_loads(0)
            pl.when(kp > 0)(lambda: wait_scatter(1))
            start_loads(kk + 1, 1, mr)
            compute_scatter(kk, 0)
            wait_loads(1)
            wait_scatter(0)
            pl.when(kp < SCHUNK // 2 - 1)(lambda: start_loads(kk + 2, 0, mr))
            compute_scatter(kk + 1, 1)
            return c1

        lax.fori_loop(0, SCHUNK // 2, pair, 0)
        wait_scatter(1)

    plsc.subcore_barrier()
    for j in range(ROWS_PER_TILE // CHUNK):
        r0 = s * ROWS_PER_TILE + j * CHUNK
        pltpu.sync_copy(acc.at[pl.ds(r0, CHUNK)], m0)
        pltpu.sync_copy(m0, out2.at[c, pl.ds(r0, CHUNK)])


def _sc_scatter(mraws, bond_weights, src_r, u_r):
    mesh = plsc.VectorSubcoreMesh(core_axis_name="c", subcore_axis_name="s")
    f = pl.kernel(
        _scatter_body,
        out_type=jax.ShapeDtypeStruct((NC, ACC_ROWS, ATOM_DIM), jnp.float32),
        mesh=mesh,
        scratch_types=[
            pltpu.VMEM((SCHUNK, CHUNK), jnp.int32),
            pltpu.VMEM((SCHUNK, CHUNK), jnp.int32),
            pltpu.VMEM((CHUNK, ATOM_DIM), jnp.float32),
            pltpu.VMEM((CHUNK, ATOM_DIM), jnp.float32),
            pltpu.VMEM((CHUNK, ATOM_DIM), jnp.float32),
            pltpu.VMEM((CHUNK, ATOM_DIM), jnp.float32),
            pltpu.VMEM_SHARED((ACC_ROWS, ATOM_DIM), jnp.float32),
            pltpu.SemaphoreType.DMA,
            pltpu.SemaphoreType.DMA,
            pltpu.SemaphoreType.DMA,
            pltpu.SemaphoreType.DMA,
        ],
    )
    return f(*mraws, bond_weights, src_r, u_r)


def _prep_body(a_ref, wa_ref, wn_ref, oa_ref, on_ref):
    a = a_ref[...]
    oa_ref[...] = jnp.dot(a, wa_ref[...], preferred_element_type=jnp.float32)
    on_ref[...] = jnp.dot(a, wn_ref[...], preferred_element_type=jnp.float32)


def _tc_prep(atom_feas, wa, wn):
    blk = 2000
    grid = N_ATOMS // blk
    return pl.pallas_call(
        _prep_body,
        grid=(grid,),
        in_specs=[
            pl.BlockSpec((blk, ATOM_DIM), lambda i: (i, 0)),
            pl.BlockSpec((ATOM_DIM, ATOM_DIM), lambda i: (0, 0)),
            pl.BlockSpec((ATOM_DIM, ATOM_DIM), lambda i: (0, 0)),
        ],
        out_specs=[
            pl.BlockSpec((blk, ATOM_DIM), lambda i: (i, 0)),
            pl.BlockSpec((blk, ATOM_DIM), lambda i: (i, 0)),
        ],
        out_shape=[
            jax.ShapeDtypeStruct((N_ATOMS, ATOM_DIM), jnp.float32),
            jax.ShapeDtypeStruct((N_ATOMS, ATOM_DIM), jnp.float32),
        ],
    )(atom_feas, wa, wn)


def _bond_body(b_ref, wb_ref, b1_ref, o_ref):
    o_ref[...] = (jnp.dot(b_ref[...], wb_ref[...],
                          preferred_element_type=jnp.float32) + b1_ref[...])


def _tc_prep_bond(bond_feas, wb, b1):
    blk = 4000
    grid = N_UND // blk
    return pl.pallas_call(
        _bond_body,
        grid=(grid,),
        in_specs=[
            pl.BlockSpec((blk, BOND_DIM), lambda i: (i, 0)),
            pl.BlockSpec((BOND_DIM, ATOM_DIM), lambda i: (0, 0)),
            pl.BlockSpec((1, ATOM_DIM), lambda i: (0, 0)),
        ],
        out_specs=pl.BlockSpec((blk, ATOM_DIM), lambda i: (i, 0)),
        out_shape=jax.ShapeDtypeStruct((N_UND, ATOM_DIM), jnp.float32),
    )(bond_feas, wb, b1)


def _mlp_body(h_ref, w2c_ref, b2c_ref, w2g_ref, b2g_ref, o_ref):
    h = h_ref[...]
    s = (h * jax.nn.sigmoid(h)).astype(jnp.bfloat16)
    core = jnp.dot(s[:, :64], w2c_ref[...],
                   preferred_element_type=jnp.float32) + b2c_ref[...]
    gate = jnp.dot(s[:, 64:], w2g_ref[...],
                   preferred_element_type=jnp.float32) + b2g_ref[...]
    o_ref[...] = core * jax.nn.sigmoid(core) * jax.nn.sigmoid(gate)


def _tc_mlp(h_sum, w2c, b2c, w2g, b2g):
    n = h_sum.shape[0]
    blk = 3200
    grid = n // blk
    return pl.pallas_call(
        _mlp_body,
        grid=(grid,),
        in_specs=[
            pl.BlockSpec((blk, ATOM_DIM), lambda i: (i, 0)),
            pl.BlockSpec((64, ATOM_DIM), lambda i: (0, 0)),
            pl.BlockSpec((1, ATOM_DIM), lambda i: (0, 0)),
            pl.BlockSpec((64, ATOM_DIM), lambda i: (0, 0)),
            pl.BlockSpec((1, ATOM_DIM), lambda i: (0, 0)),
        ],
        out_specs=pl.BlockSpec((blk, ATOM_DIM), lambda i: (i, 0)),
        out_shape=jax.ShapeDtypeStruct((n, ATOM_DIM), jnp.float32),
    )(h_sum, w2c, b2c, w2g, b2g)


def _out_body(p_ref, w_ref, b_ref, a_ref, o_ref):
    p = p_ref[0] + p_ref[1]
    o_ref[...] = (jnp.dot(p, w_ref[...], preferred_element_type=jnp.float32)
                  + b_ref[...] + a_ref[...])


def _tc_out(partials, wout, bout, atom_feas):
    blk = 2000
    grid = N_ATOMS // blk
    return pl.pallas_call(
        _out_body,
        grid=(grid,),
        in_specs=[
            pl.BlockSpec((NC, blk, ATOM_DIM), lambda i: (0, i, 0)),
            pl.BlockSpec((ATOM_DIM, ATOM_DIM), lambda i: (0, 0)),
            pl.BlockSpec((1, ATOM_DIM), lambda i: (0, 0)),
            pl.BlockSpec((blk, ATOM_DIM), lambda i: (i, 0)),
        ],
        out_specs=pl.BlockSpec((blk, ATOM_DIM), lambda i: (i, 0)),
        out_shape=jax.ShapeDtypeStruct((N_ATOMS, ATOM_DIM), jnp.float32),
    )(partials, wout, bout, atom_feas)


def kernel(atom_feas, bond_feas, bond_weights, atom_graph, directed2undirected,
           W1c, b1c, W2c, b2c, W1g, b1g, W2g, b2g, Wout, bout):
    wa = jnp.concatenate([W1c[:ATOM_DIM], W1g[:ATOM_DIM]], axis=1)
    wn = jnp.concatenate([W1c[ATOM_DIM + BOND_DIM:], W1g[ATOM_DIM + BOND_DIM:]], axis=1)
    wb = jnp.concatenate([W1c[ATOM_DIM:ATOM_DIM + BOND_DIM],
                          W1g[ATOM_DIM:ATOM_DIM + BOND_DIM]], axis=1)
    b1 = jnp.concatenate([b1c, b1g]).reshape(1, ATOM_DIM)

    src = atom_graph[:, 0]
    dst = atom_graph[:, 1]
    src_r = src.reshape(NW, NSEC, SCHUNK, CHUNK)
    u_r = directed2undirected.reshape(NW, NSEC, SCHUNK, CHUNK)
    src_g = src.reshape(NW, NSLICE, GSCH, GCH)
    dst_g = dst.reshape(NW, NSLICE, GSCH, GCH)
    u_g = directed2undirected.reshape(NW, NSLICE, GSCH, GCH)

    acat, ncat = _tc_prep(atom_feas, wa, wn)
    bcat = _tc_prep_bond(bond_feas, wb, b1)
    w2cb = W2c.astype(jnp.bfloat16)
    w2gb = W2g.astype(jnp.bfloat16)
    b2cr = b2c.reshape(1, -1)
    b2gr = b2g.reshape(1, -1)
    mraws = []
    for si in range(NSLICE):
        h_si = _sc_gather(acat, ncat, bcat,
                          src_g[:, si:si + 1], dst_g[:, si:si + 1],
                          u_g[:, si:si + 1])
        mraws.append(_tc_mlp(h_si, w2cb, b2cr, w2gb, b2gr))
    partials = _sc_scatter(mraws, bond_weights, src_r, u_r)
    return _tc_out(partials, Wout, bout.reshape(1, -1), atom_feas)
